# initial kernel scaffold (unmeasured)
import jax
import jax.numpy as jnp
from jax import lax
from jax.experimental import pallas as pl
from jax.experimental.pallas import tpu as pltpu


def kernel(
    x,
):
    def body(*refs):
        pass

    out_shape = jax.ShapeDtypeStruct(..., jnp.float32)
    return pl.pallas_call(body, out_shape=out_shape)(...)


# baseline (device time: 511632 ns/iter reference)
import jax
import jax.numpy as jnp
from jax import lax
from jax.experimental import pallas as pl
from jax.experimental.pallas import tpu as pltpu

N_DEV = 16


def kernel(x):
    m, n = x.shape
    cm = m // N_DEV

    def body(x_ref, out_ref, comm_ref, send_sem, recv_sem, credit_sem):
        my = lax.axis_index("i")
        left = lax.rem(my + N_DEV - 1, N_DEV)
        right = lax.rem(my + 1, N_DEV)

        barrier = pltpu.get_barrier_semaphore()
        for nbr in (left, right):
            pl.semaphore_signal(
                barrier, inc=1, device_id=(nbr,),
                device_id_type=pl.DeviceIdType.MESH,
            )
        pl.semaphore_wait(barrier, 2)

        def x_chunk(c):
            return x_ref[pl.ds(c * cm, cm), :]

        def hop():
            pl.semaphore_signal(
                credit_sem, inc=1, device_id=(left,),
                device_id_type=pl.DeviceIdType.MESH,
            )
            pl.semaphore_wait(credit_sem, 1)
            rdma = pltpu.make_async_remote_copy(
                src_ref=comm_ref.at[0],
                dst_ref=comm_ref.at[1],
                send_sem=send_sem,
                recv_sem=recv_sem,
                device_id=(right,),
                device_id_type=pl.DeviceIdType.MESH,
            )
            rdma.start()
            rdma.wait()

        for s in range(N_DEV - 1):
            c_send = lax.rem(my + N_DEV - s, N_DEV)
            if s == 0:
                comm_ref[0, :, :] = x_chunk(c_send)
            else:
                comm_ref[0, :, :] = comm_ref[1, :, :] + x_chunk(c_send)
            hop()

        g = lax.rem(my + 1, N_DEV)
        red = comm_ref[1, :, :] + x_chunk(g)
        out_ref[pl.ds(g * cm, cm), :] = red

        for t in range(N_DEV - 1):
            if t == 0:
                comm_ref[0, :, :] = red
            else:
                comm_ref[0, :, :] = comm_ref[1, :, :]
            hop()
            c_recv = lax.rem(my + N_DEV - t, N_DEV)
            out_ref[pl.ds(c_recv * cm, cm), :] = comm_ref[1, :, :]

    return pl.pallas_call(
        body,
        out_shape=jax.ShapeDtypeStruct((m, n), jnp.float32),
        in_specs=[pl.BlockSpec(memory_space=pltpu.VMEM)],
        out_specs=pl.BlockSpec(memory_space=pltpu.VMEM),
        scratch_shapes=[
            pltpu.VMEM((2, cm, n), jnp.float32),
            pltpu.SemaphoreType.DMA,
            pltpu.SemaphoreType.DMA,
            pltpu.SemaphoreType.REGULAR,
        ],
        compiler_params=pltpu.CompilerParams(collective_id=0),
    )(x)


# device time: 242403 ns/iter; 2.1107x vs baseline; 2.1107x over previous
import jax
import jax.numpy as jnp
from jax import lax
from jax.experimental import pallas as pl
from jax.experimental.pallas import tpu as pltpu

N_DEV = 16
N_HOPS = 2 * (N_DEV - 1)


def kernel(x):
    m, n = x.shape
    cm = m // N_DEV
    hm = cm // 2

    def body(x_ref, out_ref, comm_ref, send_sem, recv_sem, credit_sem):
        my = lax.axis_index("i")
        left = lax.rem(my + N_DEV - 1, N_DEV)
        right = lax.rem(my + 1, N_DEV)

        barrier = pltpu.get_barrier_semaphore()
        for nbr in (left, right):
            pl.semaphore_signal(
                barrier, inc=1, device_id=(nbr,),
                device_id_type=pl.DeviceIdType.MESH,
            )
        pl.semaphore_wait(barrier, 2)

        dirs = (
            {"dst": right, "up": left, "sgn": -1, "off": 0},
            {"dst": left, "up": right, "sgn": +1, "off": hm},
        )

        def chunk_rows(k, off):
            c = lax.rem(my + k + 2 * N_DEV, N_DEV)
            return pl.ds(c * cm + off, hm)

        def descriptor(di, p):
            return pltpu.make_async_remote_copy(
                src_ref=comm_ref.at[di, 0],
                dst_ref=comm_ref.at[di, 1 + p],
                send_sem=send_sem.at[di],
                recv_sem=recv_sem.at[di, p],
                device_id=(dirs[di]["dst"],),
                device_id_type=pl.DeviceIdType.MESH,
            )

        prev = [None, None]
        for h in range(N_HOPS):
            p = h % 2
            pp = (h - 1) % 2
            for di, dd in enumerate(dirs):
                sgn, off = dd["sgn"], dd["off"]
                if h >= 1:
                    descriptor(di, pp).wait_recv()
                    prev[di].wait_send()
                if h == 0:
                    comm_ref[di, 0, :, :] = x_ref[chunk_rows(0, off), :]
                elif h < N_DEV - 1:
                    comm_ref[di, 0, :, :] = (
                        comm_ref[di, 1 + pp, :, :]
                        + x_ref[chunk_rows(sgn * h, off), :]
                    )
                elif h == N_DEV - 1:
                    red = (
                        comm_ref[di, 1 + pp, :, :]
                        + x_ref[chunk_rows(-sgn, off), :]
                    )
                    out_ref[chunk_rows(-sgn, off), :] = red
                    comm_ref[di, 0, :, :] = red
                else:
                    out_ref[chunk_rows(sgn * (h - N_DEV), off), :] = (
                        comm_ref[di, 1 + pp, :, :]
                    )
                    comm_ref[di, 0, :, :] = comm_ref[di, 1 + pp, :, :]
                if 1 <= h <= N_HOPS - 2:
                    pl.semaphore_signal(
                        credit_sem.at[di], inc=1, device_id=(dd["up"],),
                        device_id_type=pl.DeviceIdType.MESH,
                    )
                if h >= 2:
                    pl.semaphore_wait(credit_sem.at[di], 1)
                rdma = descriptor(di, p)
                rdma.start()
                prev[di] = rdma

        p = (N_HOPS - 1) % 2
        for di, dd in enumerate(dirs):
            descriptor(di, p).wait_recv()
            out_ref[chunk_rows(dd["sgn"] * (N_DEV - 2), dd["off"]), :] = (
                comm_ref[di, 1 + p, :, :]
            )
            prev[di].wait_send()

    return pl.pallas_call(
        body,
        out_shape=jax.ShapeDtypeStruct((m, n), jnp.float32),
        in_specs=[pl.BlockSpec(memory_space=pltpu.VMEM)],
        out_specs=pl.BlockSpec(memory_space=pltpu.VMEM),
        scratch_shapes=[
            pltpu.VMEM((2, 3, hm, n), jnp.float32),
            pltpu.SemaphoreType.DMA((2,)),
            pltpu.SemaphoreType.DMA((2, 2)),
            pltpu.SemaphoreType.REGULAR((2,)),
        ],
        compiler_params=pltpu.CompilerParams(collective_id=0),
    )(x)


# device time: 188328 ns/iter; 2.7167x vs baseline; 1.2871x over previous
import jax
import jax.numpy as jnp
from jax import lax
from jax.experimental import pallas as pl
from jax.experimental.pallas import tpu as pltpu

N_DEV = 16
N_HOPS = 2 * (N_DEV - 1)
NSUB = 2


def kernel(x):
    m, n = x.shape
    cm = m // N_DEV
    sm = cm // (2 * NSUB)

    def body(x_ref, out_ref, comm_ref, send_sem, recv_sem, credit_sem):
        my = lax.axis_index("i")
        left = lax.rem(my + N_DEV - 1, N_DEV)
        right = lax.rem(my + 1, N_DEV)

        barrier = pltpu.get_barrier_semaphore()
        for nbr in (left, right):
            pl.semaphore_signal(
                barrier, inc=1, device_id=(nbr,),
                device_id_type=pl.DeviceIdType.MESH,
            )
        pl.semaphore_wait(barrier, 2)

        streams = []
        for si in range(NSUB):
            for di, (dst, up, sgn) in enumerate(
                ((right, left, -1), (left, right, +1))
            ):
                streams.append({
                    "dst": dst, "up": up, "sgn": sgn,
                    "off": di * (cm // 2) + si * sm,
                })
        n_str = len(streams)

        def chunk_rows(k, off):
            c = lax.rem(my + k + 2 * N_DEV, N_DEV)
            return pl.ds(c * cm + off, sm)

        def descriptor(st, p):
            return pltpu.make_async_remote_copy(
                src_ref=comm_ref.at[st, 0],
                dst_ref=comm_ref.at[st, 1 + p],
                send_sem=send_sem.at[st],
                recv_sem=recv_sem.at[st, p],
                device_id=(streams[st]["dst"],),
                device_id_type=pl.DeviceIdType.MESH,
            )

        prev = [None] * n_str
        for h in range(N_HOPS):
            p = h % 2
            pp = (h - 1) % 2
            for st, dd in enumerate(streams):
                sgn, off = dd["sgn"], dd["off"]
                if h >= 1:
                    descriptor(st, pp).wait_recv()
                    prev[st].wait_send()
                if h == 0:
                    comm_ref[st, 0, :, :] = x_ref[chunk_rows(0, off), :]
                elif h < N_DEV - 1:
                    comm_ref[st, 0, :, :] = (
                        comm_ref[st, 1 + pp, :, :]
                        + x_ref[chunk_rows(sgn * h, off), :]
                    )
                elif h == N_DEV - 1:
                    red = (
                        comm_ref[st, 1 + pp, :, :]
                        + x_ref[chunk_rows(-sgn, off), :]
                    )
                    out_ref[chunk_rows(-sgn, off), :] = red
                    comm_ref[st, 0, :, :] = red
                else:
                    out_ref[chunk_rows(sgn * (h - N_DEV), off), :] = (
                        comm_ref[st, 1 + pp, :, :]
                    )
                    comm_ref[st, 0, :, :] = comm_ref[st, 1 + pp, :, :]
                if 1 <= h <= N_HOPS - 2:
                    pl.semaphore_signal(
                        credit_sem.at[st], inc=1, device_id=(dd["up"],),
                        device_id_type=pl.DeviceIdType.MESH,
                    )
                if h >= 2:
                    pl.semaphore_wait(credit_sem.at[st], 1)
                rdma = descriptor(st, p)
                rdma.start()
                prev[st] = rdma

        p = (N_HOPS - 1) % 2
        for st, dd in enumerate(streams):
            descriptor(st, p).wait_recv()
            out_ref[chunk_rows(dd["sgn"] * (N_DEV - 2), dd["off"]), :] = (
                comm_ref[st, 1 + p, :, :]
            )
            prev[st].wait_send()

    n_str = 2 * NSUB
    return pl.pallas_call(
        body,
        out_shape=jax.ShapeDtypeStruct((m, n), jnp.float32),
        in_specs=[pl.BlockSpec(memory_space=pltpu.VMEM)],
        out_specs=pl.BlockSpec(memory_space=pltpu.VMEM),
        scratch_shapes=[
            pltpu.VMEM((n_str, 3, sm, n), jnp.float32),
            pltpu.SemaphoreType.DMA((n_str,)),
            pltpu.SemaphoreType.DMA((n_str, 2)),
            pltpu.SemaphoreType.REGULAR((n_str,)),
        ],
        compiler_params=pltpu.CompilerParams(collective_id=0),
    )(x)
